# baseline (speedup 1.0000x reference)
import functools

import jax
import jax.numpy as jnp
from jax import lax
from jax.experimental import pallas as pl
from jax.experimental.pallas import tpu as pltpu
from jax.experimental.pallas import tpu_sc as plsc

NC = 2
NS = 16
NW = NC * NS

SUB = 128
K = 8
CHUNK = K * SUB


def _gather_body(idx_hbm, table_hbm, out_hbm, idx_v, rows_v, sem):
    n_rows = idx_hbm.shape[0]
    per_w_rows = n_rows // NW
    n_chunks = per_w_rows // K
    wid = lax.axis_index("s") * NC + lax.axis_index("c")
    row0 = wid * per_w_rows

    def chunk(ci, carry):
        r = row0 + ci * K
        pltpu.sync_copy(idx_hbm.at[pl.ds(r, K)], idx_v)
        copies = [
            pltpu.async_copy(
                table_hbm.at[idx_v.at[j]],
                rows_v.at[pl.ds(j * SUB, SUB)],
                sem,
            )
            for j in range(K)
        ]
        for c in copies:
            c.wait()
        pltpu.sync_copy(rows_v, out_hbm.at[pl.ds(r * SUB, CHUNK)])
        return carry

    lax.fori_loop(0, n_chunks, chunk, 0)


def kernel(x, embedding):
    B, L = x.shape
    D = embedding.shape[1]
    n = B * L
    assert n % (NW * CHUNK) == 0
    idx = x.reshape(n // SUB, SUB).astype(jnp.int32)

    mesh = plsc.VectorSubcoreMesh(core_axis_name="c", subcore_axis_name="s")
    run = pl.kernel(
        _gather_body,
        out_type=jax.ShapeDtypeStruct((n, D), jnp.float32),
        mesh=mesh,
        scratch_types=[
            pltpu.VMEM((K, SUB), jnp.int32),
            pltpu.VMEM((CHUNK, D), jnp.float32),
            pltpu.SemaphoreType.DMA,
        ],
        compiler_params=pltpu.CompilerParams(use_tc_tiling_on_sc=False),
    )
    out = run(idx, embedding)
    return out.reshape(B, L, D)


# speedup vs baseline: 1.0327x; 1.0327x over previous
import jax
import jax.numpy as jnp
from jax import lax
from jax.experimental import pallas as pl
from jax.experimental.pallas import tpu as pltpu
from jax.experimental.pallas import tpu_sc as plsc

NC = 2
NS = 16
NW = NC * NS

SUB = 128
K = 5
CHUNK = K * SUB


def _gather_body(idx_hbm, table_hbm, out_hbm, idx_v, rows_v, sem_g, sem_o,
                 sem_i):
    n_rows = idx_hbm.shape[0]
    per_w_rows = n_rows // NW
    n_chunks = per_w_rows // K
    wid = lax.axis_index("s") * NC + lax.axis_index("c")
    row0 = wid * per_w_rows

    def issue_gathers(p, _):
        for j in range(K):
            pltpu.async_copy(
                table_hbm.at[idx_v.at[p].at[j]],
                rows_v.at[p].at[pl.ds(j * SUB, SUB)],
                sem_g,
            )

    def drain_gathers(p):
        pltpu.make_async_copy(
            table_hbm.at[pl.ds(0, CHUNK)], rows_v.at[p], sem_g
        ).wait()

    def fetch_idx(ci, p):
        r = row0 + lax.rem(ci, n_chunks) * K
        return pltpu.async_copy(idx_hbm.at[pl.ds(r, K)], idx_v.at[p], sem_i)

    def wait_idx(p):
        pltpu.make_async_copy(
            idx_hbm.at[pl.ds(0, K)], idx_v.at[p], sem_i
        ).wait()

    def issue_write(ci, p):
        start = (row0 + ci * K) * SUB
        return pltpu.async_copy(
            rows_v.at[p], out_hbm.at[pl.ds(start, CHUNK)], sem_o
        )

    def wait_write(p):
        pltpu.make_async_copy(
            rows_v.at[p], out_hbm.at[pl.ds(0, CHUNK)], sem_o
        ).wait()

    pltpu.sync_copy(idx_hbm.at[pl.ds(row0, K)], idx_v.at[0])
    issue_gathers(0, None)
    fetch_idx(1, 1)

    def body(i, carry):
        p = lax.rem(i, 2)
        q = 1 - p
        drain_gathers(p)

        @pl.when(i > 0)
        def _():
            wait_write(q)

        issue_write(i, p)
        wait_idx(q)
        fetch_idx(i + 2, p)
        issue_gathers(q, None)
        return carry

    lax.fori_loop(0, n_chunks - 1, body, 0)

    last = n_chunks - 1
    p = lax.rem(last, 2)
    drain_gathers(p)
    wait_write(1 - p)
    issue_write(last, p)
    wait_idx(1 - p)
    wait_write(p)


def kernel(x, embedding):
    B, L = x.shape
    D = embedding.shape[1]
    n = B * L
    assert n % (NW * CHUNK) == 0
    idx = x.reshape(n // SUB, SUB).astype(jnp.int32)

    mesh = plsc.VectorSubcoreMesh(core_axis_name="c", subcore_axis_name="s")
    run = pl.kernel(
        _gather_body,
        out_type=jax.ShapeDtypeStruct((n, D), jnp.float32),
        mesh=mesh,
        scratch_types=[
            pltpu.VMEM((2, K, SUB), jnp.int32),
            pltpu.VMEM((2, CHUNK, D), jnp.float32),
            pltpu.SemaphoreType.DMA,
            pltpu.SemaphoreType.DMA,
            pltpu.SemaphoreType.DMA,
        ],
        compiler_params=pltpu.CompilerParams(use_tc_tiling_on_sc=False),
    )
    out = run(idx, embedding)
    return out.reshape(B, L, D)


# speedup vs baseline: 1.0341x; 1.0013x over previous
import jax
import jax.numpy as jnp
from jax import lax
from jax.experimental import pallas as pl
from jax.experimental.pallas import tpu as pltpu
from jax.experimental.pallas import tpu_sc as plsc

NC = 2
NS = 16
NW = NC * NS

RB = 4


def _gather_body(idx_hbm, table_hbm, out_hbm, idx_v, rows_v, sem_g, sem_o,
                 sem_i):
    n_b = idx_hbm.shape[0]
    per_w = n_b // NW
    n_chunks = per_w // RB
    wid = lax.axis_index("s") * NC + lax.axis_index("c")
    b0 = wid * per_w

    def issue_gathers(p, _):
        for rb in range(RB):
            pltpu.async_copy(
                table_hbm.at[idx_v.at[p].at[rb]],
                rows_v.at[p].at[rb],
                sem_g,
            )

    def drain_gathers(p):
        pltpu.make_async_copy(
            out_hbm.at[pl.ds(0, RB)], rows_v.at[p], sem_g
        ).wait()

    def fetch_idx(ci, p):
        b = b0 + lax.rem(ci, n_chunks) * RB
        return pltpu.async_copy(idx_hbm.at[pl.ds(b, RB)], idx_v.at[p], sem_i)

    def wait_idx(p):
        pltpu.make_async_copy(
            idx_hbm.at[pl.ds(0, RB)], idx_v.at[p], sem_i
        ).wait()

    def issue_write(ci, p):
        b = b0 + ci * RB
        return pltpu.async_copy(
            rows_v.at[p], out_hbm.at[pl.ds(b, RB)], sem_o
        )

    def wait_write(p):
        pltpu.make_async_copy(
            rows_v.at[p], out_hbm.at[pl.ds(0, RB)], sem_o
        ).wait()

    pltpu.sync_copy(idx_hbm.at[pl.ds(b0, RB)], idx_v.at[0])
    issue_gathers(0, None)
    fetch_idx(1, 1)

    def body(i, carry):
        p = lax.rem(i, 2)
        q = 1 - p
        drain_gathers(p)

        @pl.when(i > 0)
        def _():
            wait_write(q)

        issue_write(i, p)
        wait_idx(q)
        fetch_idx(i + 2, p)
        issue_gathers(q, None)
        return carry

    lax.fori_loop(0, n_chunks - 1, body, 0)

    last = n_chunks - 1
    p = lax.rem(last, 2)
    drain_gathers(p)
    wait_write(1 - p)
    issue_write(last, p)
    wait_idx(1 - p)
    wait_write(p)


def kernel(x, embedding):
    B, L = x.shape
    D = embedding.shape[1]
    assert B % (NW * RB) == 0
    idx = x.astype(jnp.int32)

    mesh = plsc.VectorSubcoreMesh(core_axis_name="c", subcore_axis_name="s")
    run = pl.kernel(
        _gather_body,
        out_type=jax.ShapeDtypeStruct((B, L, D), jnp.float32),
        mesh=mesh,
        scratch_types=[
            pltpu.VMEM((2, RB, L), jnp.int32),
            pltpu.VMEM((2, RB, L, D), jnp.float32),
            pltpu.SemaphoreType.DMA,
            pltpu.SemaphoreType.DMA,
            pltpu.SemaphoreType.DMA,
        ],
        compiler_params=pltpu.CompilerParams(use_tc_tiling_on_sc=False),
    )
    return run(idx, embedding)


# speedup vs baseline: 1.2636x; 1.2220x over previous
import jax
import jax.numpy as jnp
from jax import lax
from jax.experimental import pallas as pl
from jax.experimental.pallas import tpu as pltpu
from jax.experimental.pallas import tpu_sc as plsc

NC = 2
NS = 16
NW = NC * NS


def _gather_body(idx_hbm, table_hbm, out_hbm, idx_v, rows_v, sem_g, sem_o,
                 sem_i):
    L, B = idx_hbm.shape
    D = table_hbm.shape[1]
    CB = B // NW
    wid = lax.axis_index("s") * NC + lax.axis_index("c")
    b0 = wid * CB

    def issue_gather(p, _):
        pltpu.async_copy(table_hbm.at[idx_v.at[p]], rows_v.at[p], sem_g)

    def drain_gather(p):
        pltpu.make_async_copy(
            table_hbm.at[pl.ds(0, CB)], rows_v.at[p], sem_g
        ).wait()

    def fetch_idx(l, p):
        return pltpu.async_copy(
            idx_hbm.at[lax.rem(l, L)].at[pl.ds(b0, CB)], idx_v.at[p], sem_i
        )

    def wait_idx(p):
        pltpu.make_async_copy(
            idx_hbm.at[0].at[pl.ds(0, CB)], idx_v.at[p], sem_i
        ).wait()

    def issue_write(l, p):
        return pltpu.async_copy(
            rows_v.at[p],
            out_hbm.at[pl.ds(b0, CB), pl.ds(l * D, D)],
            sem_o,
        )

    def wait_write(p):
        pltpu.make_async_copy(
            rows_v.at[p], out_hbm.at[pl.ds(0, CB), pl.ds(0, D)], sem_o
        ).wait()

    pltpu.sync_copy(idx_hbm.at[0].at[pl.ds(b0, CB)], idx_v.at[0])
    issue_gather(0, None)
    fetch_idx(1, 1)

    def body(l, carry):
        p = lax.rem(l, 2)
        q = 1 - p
        drain_gather(p)

        @pl.when(l > 0)
        def _():
            wait_write(q)

        issue_write(l, p)
        wait_idx(q)
        fetch_idx(l + 2, p)
        issue_gather(q, None)
        return carry

    lax.fori_loop(0, L - 1, body, 0)

    last = L - 1
    p = lax.rem(last, 2)
    drain_gather(p)
    wait_write(1 - p)
    issue_write(last, p)
    wait_idx(1 - p)
    wait_write(p)


def kernel(x, embedding):
    B, L = x.shape
    D = embedding.shape[1]
    assert B % NW == 0
    CB = B // NW
    idx_t = x.T.astype(jnp.int32)

    mesh = plsc.VectorSubcoreMesh(core_axis_name="c", subcore_axis_name="s")
    run = pl.kernel(
        _gather_body,
        out_type=jax.ShapeDtypeStruct((B, L * D), jnp.float32),
        mesh=mesh,
        scratch_types=[
            pltpu.VMEM((2, CB), jnp.int32),
            pltpu.VMEM((2, CB, D), jnp.float32),
            pltpu.SemaphoreType.DMA,
            pltpu.SemaphoreType.DMA,
            pltpu.SemaphoreType.DMA,
        ],
        compiler_params=pltpu.CompilerParams(use_tc_tiling_on_sc=False),
    )
    out = run(idx_t, embedding)
    return out.reshape(B, L, D)
